# Initial kernel scaffold; baseline (speedup 1.0000x reference)
import jax
import jax.numpy as jnp
from jax.experimental import pallas as pl


def kernel(Z, W, neural_atom_idx, atom_obj_idx, atom_pred_idx, bg_atom_idx, n_atoms):
    raise NotImplementedError("write your pallas kernel here")


# speedup vs baseline: 6.1674x; 6.1674x over previous
import functools

import jax
import jax.numpy as jnp
from jax import lax
from jax.experimental import pallas as pl
from jax.experimental.pallas import tpu as pltpu
from jax.experimental.pallas import tpu_sc as plsc

B = 16
N_OBJ = 128
N_PRED = 32
FEAT = 64
N_ATOMS = 500000
N_NEURAL = 200000
N_BG = 50000

NC = 2
NS = 16
NW = NC * NS

TW = 4104
ONE_SLOT = 4096
ZERO_SLOT = 4097
TABLE_N = B * TW

ADDR_N = 501760
DUMP = 500000
INIT_CHUNK = 2048
N_INIT_CHUNKS = 245

NEUR_PT = 12800
BG_PT = 3200

CH = 800
N_CHUNKS = 625
MAX_CH_PER_TILE = 20


def _tc_table(z2, w):
    def body(z_ref, w_ref, o_ref):
        s = lax.dot_general(z_ref[...], w_ref[...], (((1,), (1,)), ((), ())),
                            preferred_element_type=jnp.float32)
        o_ref[...] = jax.nn.sigmoid(s)
    return pl.pallas_call(
        body,
        out_shape=jax.ShapeDtypeStruct((B * N_OBJ, N_PRED), jnp.float32),
    )(z2, w)


def _sc_build(table, nidx, comb_src_obj, comb_src_pred, bg):
    i32 = jnp.int32
    mesh = plsc.VectorSubcoreMesh(core_axis_name="c", subcore_axis_name="s",
                                  num_cores=NC, num_subcores=NS)

    @functools.partial(
        pl.kernel,
        out_type=jax.ShapeDtypeStruct((B * N_ATOMS,), jnp.float32),
        mesh=mesh,
        scratch_types=[
            pltpu.VMEM_SHARED((ADDR_N,), i32),
        ],
        compiler_params=pltpu.CompilerParams(needs_layout_passes=False),
    )
    def body(table_h, nidx_h, obj_h, prd_h, bg_h, out_h, addr_sh):
        c = lax.axis_index("c")
        s = lax.axis_index("s")
        wid = s * NC + c

        def scatter_scope(nidx_v, comb_v, prd_v, bgidx_v, bgval_v):
            zsplat = jnp.full((16,), ZERO_SLOT, i32)
            def fill_body(i, carry):
                comb_v[pl.ds(i * 16, 16)] = zsplat
                return carry
            lax.fori_loop(0, INIT_CHUNK // 16, fill_body, 0)

            def init_body(i, carry):
                cid = s + NS * i
                @pl.when(cid < N_INIT_CHUNKS)
                def _():
                    pltpu.sync_copy(comb_v.at[pl.ds(0, INIT_CHUNK)],
                                    addr_sh.at[pl.ds(cid * INIT_CHUNK,
                                                     INIT_CHUNK)])
                return carry
            lax.fori_loop(0, 16, init_body, 0)
            plsc.subcore_barrier()

            pltpu.sync_copy(nidx_h.at[pl.ds(s * NEUR_PT, NEUR_PT)], nidx_v)
            pltpu.sync_copy(obj_h.at[pl.ds(s * NEUR_PT, NEUR_PT)], comb_v)
            pltpu.sync_copy(prd_h.at[pl.ds(s * NEUR_PT, NEUR_PT)], prd_v)

            def comb_body(g, carry):
                o = comb_v[pl.ds(g * 16, 16)]
                p = prd_v[pl.ds(g * 16, 16)]
                comb_v[pl.ds(g * 16, 16)] = o * N_PRED + p
                return carry
            lax.fori_loop(0, NEUR_PT // 16, comb_body, 0)

            pltpu.sync_copy(bg_h.at[pl.ds(s * BG_PT, BG_PT)], bgidx_v)
            osplat = jnp.full((16,), ONE_SLOT, i32)
            def bg_body(g, carry):
                bgval_v[pl.ds(g * 16, 16)] = osplat
                return carry
            lax.fori_loop(0, BG_PT // 16, bg_body, 0)

            pltpu.sync_copy(comb_v, addr_sh.at[nidx_v])
            pltpu.sync_copy(bgval_v, addr_sh.at[bgidx_v])

        pl.run_scoped(scatter_scope,
                      pltpu.VMEM((NEUR_PT,), i32),
                      pltpu.VMEM((NEUR_PT,), i32),
                      pltpu.VMEM((NEUR_PT,), i32),
                      pltpu.VMEM((BG_PT,), i32),
                      pltpu.VMEM((BG_PT,), i32))
        plsc.subcore_barrier()

        def dense_scope(table_v, addr_v, out_v):
            pltpu.sync_copy(table_h, table_v)

            def chunk_body(i, carry):
                cid = wid + NW * i
                @pl.when(cid < N_CHUNKS)
                def _():
                    cbase = cid * CH
                    pltpu.sync_copy(addr_sh.at[pl.ds(cbase, CH)], addr_v)

                    def g_body(g, carry2):
                        a = addr_v[pl.ds(g * 16, 16)]
                        for b in range(B):
                            fi = a + (b * TW)
                            v = plsc.load_gather(table_v, [fi])
                            out_v[pl.ds(b * CH + g * 16, 16)] = v
                        return carry2
                    lax.fori_loop(0, CH // 16, g_body, 0)

                    for b in range(B):
                        pltpu.sync_copy(out_v.at[pl.ds(b * CH, CH)],
                                        out_h.at[pl.ds(b * N_ATOMS + cbase,
                                                       CH)])
                return carry
            lax.fori_loop(0, MAX_CH_PER_TILE, chunk_body, 0)

        pl.run_scoped(dense_scope,
                      pltpu.VMEM((TABLE_N,), jnp.float32),
                      pltpu.VMEM((CH,), i32),
                      pltpu.VMEM((B * CH,), jnp.float32))

    return body(table, nidx, comb_src_obj, comb_src_pred, bg)


def kernel(Z, W, neural_atom_idx, atom_obj_idx, atom_pred_idx, bg_atom_idx,
           n_atoms):
    i32 = jnp.int32
    del n_atoms

    scores = _tc_table(Z.reshape(B * N_OBJ, FEAT), W)
    tail = jnp.tile(
        jnp.array([[1.0] + [0.0] * (TW - N_OBJ * N_PRED - 1)], jnp.float32),
        (B, 1))
    table = jnp.concatenate([scores.reshape(B, N_OBJ * N_PRED), tail],
                            axis=1).reshape(-1)

    npad = NEUR_PT * NS - N_NEURAL
    nidx = jnp.concatenate(
        [neural_atom_idx, jnp.full((npad,), DUMP, i32)])
    obj = jnp.concatenate([atom_obj_idx, jnp.zeros((npad,), i32)])
    prd = jnp.concatenate([atom_pred_idx, jnp.zeros((npad,), i32)])
    bgpad = BG_PT * NS - N_BG - 1
    bg = jnp.concatenate(
        [bg_atom_idx, jnp.array([1], i32), jnp.full((bgpad,), DUMP, i32)])

    return _sc_build(table, nidx, obj, prd, bg).reshape(B, N_ATOMS)


# speedup vs baseline: 52.8364x; 8.5671x over previous
import functools

import jax
import jax.numpy as jnp
from jax import lax
from jax.experimental import pallas as pl
from jax.experimental.pallas import tpu as pltpu
from jax.experimental.pallas import tpu_sc as plsc

B = 16
N_OBJ = 128
N_PRED = 32
FEAT = 64
N_ATOMS = 500000
N_NEURAL = 200000
N_BG = 50000

NC = 2
NS = 16
NW = NC * NS

TW = 4104
ONE_SLOT = 4096
ZERO_SLOT = 4097
TABLE_N = B * TW

ADDR_N = 501760
DUMP = 500000
INIT_CHUNK = 2048
N_INIT_CHUNKS = 245

NEUR_PT = 12800
BG_PT = 3200

CH = 768
N_FULL = 651
TAIL = 32
MAX_CH_PER_TILE = 21


def _tc_table(z2, w):
    def body(z_ref, w_ref, o_ref):
        s = lax.dot_general(z_ref[...], w_ref[...], (((1,), (1,)), ((), ())),
                            preferred_element_type=jnp.float32)
        o_ref[...] = jax.nn.sigmoid(s)
    return pl.pallas_call(
        body,
        out_shape=jax.ShapeDtypeStruct((B * N_OBJ, N_PRED), jnp.float32),
    )(z2, w)


def _sc_build(table, nidx, comb_src_obj, comb_src_pred, bg):
    i32 = jnp.int32
    mesh = plsc.VectorSubcoreMesh(core_axis_name="c", subcore_axis_name="s",
                                  num_cores=NC, num_subcores=NS)

    @functools.partial(
        pl.kernel,
        out_type=jax.ShapeDtypeStruct((B, N_ATOMS), jnp.float32),
        mesh=mesh,
        scratch_types=[
            pltpu.VMEM_SHARED((ADDR_N,), i32),
        ],
        compiler_params=pltpu.CompilerParams(needs_layout_passes=False),
    )
    def body(table_h, nidx_h, obj_h, prd_h, bg_h, out_h, addr_sh):
        c = lax.axis_index("c")
        s = lax.axis_index("s")
        wid = s * NC + c

        def scatter_scope(nidx_v, comb_v, prd_v, bgidx_v, bgval_v):
            zsplat = jnp.full((16,), ZERO_SLOT, i32)
            def fill_body(i, carry):
                comb_v[pl.ds(i * 16, 16)] = zsplat
                return carry
            lax.fori_loop(0, INIT_CHUNK // 16, fill_body, 0)

            def init_body(i, carry):
                cid = s + NS * i
                @pl.when(cid < N_INIT_CHUNKS)
                def _():
                    pltpu.sync_copy(comb_v.at[pl.ds(0, INIT_CHUNK)],
                                    addr_sh.at[pl.ds(cid * INIT_CHUNK,
                                                     INIT_CHUNK)])
                return carry
            lax.fori_loop(0, 16, init_body, 0)
            plsc.subcore_barrier()

            pltpu.sync_copy(nidx_h.at[pl.ds(s * NEUR_PT, NEUR_PT)], nidx_v)
            pltpu.sync_copy(obj_h.at[pl.ds(s * NEUR_PT, NEUR_PT)], comb_v)
            pltpu.sync_copy(prd_h.at[pl.ds(s * NEUR_PT, NEUR_PT)], prd_v)

            def comb_body(g, carry):
                o = comb_v[pl.ds(g * 16, 16)]
                p = prd_v[pl.ds(g * 16, 16)]
                comb_v[pl.ds(g * 16, 16)] = o * N_PRED + p
                return carry
            lax.fori_loop(0, NEUR_PT // 16, comb_body, 0)

            pltpu.sync_copy(bg_h.at[pl.ds(s * BG_PT, BG_PT)], bgidx_v)
            osplat = jnp.full((16,), ONE_SLOT, i32)
            def bg_body(g, carry):
                bgval_v[pl.ds(g * 16, 16)] = osplat
                return carry
            lax.fori_loop(0, BG_PT // 16, bg_body, 0)

            pltpu.sync_copy(comb_v, addr_sh.at[nidx_v])
            pltpu.sync_copy(bgval_v, addr_sh.at[bgidx_v])

        pl.run_scoped(scatter_scope,
                      pltpu.VMEM((NEUR_PT,), i32),
                      pltpu.VMEM((NEUR_PT,), i32),
                      pltpu.VMEM((NEUR_PT,), i32),
                      pltpu.VMEM((BG_PT,), i32),
                      pltpu.VMEM((BG_PT,), i32))
        plsc.subcore_barrier()

        def dense_scope(table_v, addr_v, out_v, tail_v):
            pltpu.sync_copy(table_h, table_v)

            def do_chunk(cbase, n, buf):
                pltpu.sync_copy(addr_sh.at[pl.ds(cbase, n)],
                                addr_v.at[pl.ds(0, n)])

                def g_body(g, carry2):
                    a = addr_v[pl.ds(g * 16, 16)]
                    for b in range(B):
                        fi = a + (b * TW)
                        v = plsc.load_gather(table_v, [fi])
                        buf[b, pl.ds(g * 16, 16)] = v
                    return carry2
                lax.fori_loop(0, n // 16, g_body, 0)

                pltpu.sync_copy(buf, out_h.at[:, pl.ds(cbase, n)])

            def chunk_body(i, carry):
                cid = wid + NW * i
                @pl.when(cid < N_FULL)
                def _():
                    do_chunk(cid * CH, CH, out_v)
                @pl.when(cid == N_FULL)
                def _():
                    do_chunk(N_FULL * CH, TAIL, tail_v)
                return carry
            lax.fori_loop(0, MAX_CH_PER_TILE, chunk_body, 0)

        pl.run_scoped(dense_scope,
                      pltpu.VMEM((TABLE_N,), jnp.float32),
                      pltpu.VMEM((CH,), i32),
                      pltpu.VMEM((B, CH), jnp.float32),
                      pltpu.VMEM((B, TAIL), jnp.float32))

    return body(table, nidx, comb_src_obj, comb_src_pred, bg)


def kernel(Z, W, neural_atom_idx, atom_obj_idx, atom_pred_idx, bg_atom_idx,
           n_atoms):
    i32 = jnp.int32
    del n_atoms

    scores = _tc_table(Z.reshape(B * N_OBJ, FEAT), W)
    tail = jnp.tile(
        jnp.array([[1.0] + [0.0] * (TW - N_OBJ * N_PRED - 1)], jnp.float32),
        (B, 1))
    table = jnp.concatenate([scores.reshape(B, N_OBJ * N_PRED), tail],
                            axis=1).reshape(-1)

    npad = NEUR_PT * NS - N_NEURAL
    nidx = jnp.concatenate(
        [neural_atom_idx, jnp.full((npad,), DUMP, i32)])
    obj = jnp.concatenate([atom_obj_idx, jnp.zeros((npad,), i32)])
    prd = jnp.concatenate([atom_pred_idx, jnp.zeros((npad,), i32)])
    bgpad = BG_PT * NS - N_BG - 1
    bg = jnp.concatenate(
        [bg_atom_idx, jnp.array([1], i32), jnp.full((bgpad,), DUMP, i32)])

    return _sc_build(table, nidx, obj, prd, bg)
